# baseline (speedup 1.0000x reference)
import functools

import jax
import jax.numpy as jnp
from jax import lax
from jax.experimental import pallas as pl
from jax.experimental.pallas import tpu as pltpu
from jax.experimental.pallas import tpu_sc as plsc

_N_TOK = 16384
_N_EMB = 8192
_D = 32
_TBLK = 512
_NB = _N_TOK // _TBLK


def _argmin_body(x_ref, emb_ref, idx_ref):
    xb = x_ref[...]
    emb = emb_ref[...]
    s = lax.dot_general(
        xb.astype(jnp.bfloat16), emb.astype(jnp.bfloat16),
        (((1,), (1,)), ((), ())),
        preferred_element_type=jnp.float32,
    )
    xsq = jnp.sum(xb * xb, axis=1, keepdims=True)
    e2 = emb * emb
    ones = jnp.ones((1, _D), jnp.float32)
    esq = lax.dot_general(
        ones, e2, (((1,), (1,)), ((), ())),
        precision=lax.Precision.HIGHEST,
        preferred_element_type=jnp.float32,
    )
    dist = xsq + esq - 2.0 * s
    mv = jnp.min(dist, axis=1, keepdims=True)
    ii = lax.broadcasted_iota(jnp.int32, (_TBLK, _N_EMB), 1)
    idx = jnp.min(jnp.where(dist == mv, ii, _N_EMB), axis=1)
    idx_ref[0, 0, :] = idx


def _encode(x_flat, embedding):
    return pl.pallas_call(
        _argmin_body,
        grid=(_NB,),
        in_specs=[
            pl.BlockSpec((_TBLK, _D), lambda i: (i, 0)),
            pl.BlockSpec((_N_EMB, _D), lambda i: (0, 0)),
        ],
        out_specs=pl.BlockSpec((1, 1, _TBLK), lambda i: (i, 0, 0)),
        out_shape=jax.ShapeDtypeStruct((_NB, 1, _TBLK), jnp.int32),
    )(x_flat, embedding)


_IDX_COLS = 128
_IDX_ROWS = _N_TOK // _IDX_COLS
_NW = 32
_ROWS_PER_W = _IDX_ROWS // _NW
_B_PER_W = _N_TOK // _NW


@functools.cache
def _make_sc_gather():
    @functools.partial(
        pl.kernel,
        out_type=jax.ShapeDtypeStruct((_N_TOK, _D), jnp.float32),
        mesh=plsc.VectorSubcoreMesh(core_axis_name="c", subcore_axis_name="s"),
        scratch_types=[
            pltpu.VMEM((_ROWS_PER_W, _IDX_COLS), jnp.int32),
            pltpu.VMEM((_B_PER_W, _D), jnp.float32),
            pltpu.SemaphoreType.DMA,
        ],
        compiler_params=pltpu.CompilerParams(use_tc_tiling_on_sc=False),
    )
    def _sc_gather(table_hbm, idx_hbm, out_hbm, idx_v, rows_v, sem):
        wid = lax.axis_index("s") * 2 + lax.axis_index("c")
        base = wid * _B_PER_W
        pltpu.sync_copy(idx_hbm.at[pl.ds(wid * _ROWS_PER_W, _ROWS_PER_W)], idx_v)
        copies = [
            pltpu.async_copy(
                table_hbm.at[idx_v.at[j]],
                rows_v.at[pl.ds(j * _IDX_COLS, _IDX_COLS)],
                sem,
            )
            for j in range(_ROWS_PER_W)
        ]
        for cp in copies:
            cp.wait()
        pltpu.sync_copy(rows_v, out_hbm.at[pl.ds(base, _B_PER_W)])

    return _sc_gather


def kernel(x, embedding):
    b, h, w, c = x.shape
    x_flat = jnp.reshape(x, (b * h * w, c))
    idx = _encode(x_flat, embedding)
    idx2 = jnp.reshape(idx, (_IDX_ROWS, _IDX_COLS))
    quantized = _make_sc_gather()(embedding, idx2)
    quantized = quantized.astype(jnp.bfloat16).astype(jnp.float32)
    return jnp.reshape(quantized, (b, h, w, c))


# speedup vs baseline: 1.6603x; 1.6603x over previous
import functools

import jax
import jax.numpy as jnp
from jax import lax
from jax.experimental import pallas as pl
from jax.experimental.pallas import tpu as pltpu
from jax.experimental.pallas import tpu_sc as plsc

_N_TOK = 16384
_N_EMB = 8192
_D = 32
_TBLK = 512
_NB = _N_TOK // _TBLK


_CH = 1024
_NCH = _N_EMB // _CH


def _argmin_body(x_ref, emb_ref, idx_ref, esq_ref, iota_ref, ebf_ref):
    @pl.when(pl.program_id(0) == 0)
    def _():
        emb = emb_ref[...]
        e2 = emb * emb
        ones = jnp.ones((1, _D), jnp.float32)
        esq_ref[...] = lax.dot_general(
            ones, e2, (((1,), (1,)), ((), ())),
            precision=lax.Precision.HIGHEST,
            preferred_element_type=jnp.float32,
        )
        iota_ref[...] = lax.broadcasted_iota(
            jnp.int32, (1, _CH), 1).astype(jnp.float32)
        ebf_ref[...] = emb.astype(jnp.bfloat16)

    xb = x_ref[...]
    xm2 = (xb * -2.0).astype(jnp.bfloat16)
    xsq = jnp.sum(xb * xb, axis=1, keepdims=True)
    iota = iota_ref[...]
    mvs, idxs = [], []
    for c in range(_NCH):
        ec = ebf_ref[pl.ds(c * _CH, _CH), :]
        s2c = lax.dot_general(
            xm2, ec, (((1,), (1,)), ((), ())),
            preferred_element_type=jnp.float32,
        )
        distc = (xsq + esq_ref[:, pl.ds(c * _CH, _CH)]) + s2c
        mvc = jnp.min(distc, axis=1, keepdims=True)
        selc = jnp.where(distc == mvc, iota, jnp.float32(_N_EMB))
        idxc = jnp.min(selc, axis=1, keepdims=True) + jnp.float32(c * _CH)
        mvs.append(mvc)
        idxs.append(idxc)
    mvs = jnp.concatenate(mvs, axis=1)
    idxs = jnp.concatenate(idxs, axis=1)
    gm = jnp.min(mvs, axis=1, keepdims=True)
    gi = jnp.min(jnp.where(mvs == gm, idxs, jnp.float32(_N_EMB)), axis=1)
    idx_ref[0, 0, :] = gi.astype(jnp.int32)


def _encode(x_flat, embedding):
    return pl.pallas_call(
        _argmin_body,
        grid=(_NB,),
        in_specs=[
            pl.BlockSpec((_TBLK, _D), lambda i: (i, 0)),
            pl.BlockSpec((_N_EMB, _D), lambda i: (0, 0)),
        ],
        out_specs=pl.BlockSpec((1, 1, _TBLK), lambda i: (i, 0, 0)),
        out_shape=jax.ShapeDtypeStruct((_NB, 1, _TBLK), jnp.int32),
        scratch_shapes=[pltpu.VMEM((1, _N_EMB), jnp.float32),
                        pltpu.VMEM((1, _CH), jnp.float32),
                        pltpu.VMEM((_N_EMB, _D), jnp.bfloat16)],
    )(x_flat, embedding)


_IDX_COLS = 128
_IDX_ROWS = _N_TOK // _IDX_COLS
_NW = 32
_ROWS_PER_W = _IDX_ROWS // _NW
_B_PER_W = _N_TOK // _NW


@functools.cache
def _make_sc_gather():
    @functools.partial(
        pl.kernel,
        out_type=jax.ShapeDtypeStruct((_N_TOK, _D), jnp.float32),
        mesh=plsc.VectorSubcoreMesh(core_axis_name="c", subcore_axis_name="s"),
        scratch_types=[
            pltpu.VMEM((_ROWS_PER_W, _IDX_COLS), jnp.int32),
            pltpu.VMEM((_B_PER_W, _D), jnp.float32),
            pltpu.SemaphoreType.DMA,
        ],
        compiler_params=pltpu.CompilerParams(use_tc_tiling_on_sc=False),
    )
    def _sc_gather(table_hbm, idx_hbm, out_hbm, idx_v, rows_v, sem):
        wid = lax.axis_index("s") * 2 + lax.axis_index("c")
        base = wid * _B_PER_W
        pltpu.sync_copy(idx_hbm.at[pl.ds(wid * _ROWS_PER_W, _ROWS_PER_W)], idx_v)
        copies = [
            pltpu.async_copy(
                table_hbm.at[idx_v.at[j]],
                rows_v.at[pl.ds(j * _IDX_COLS, _IDX_COLS)],
                sem,
            )
            for j in range(_ROWS_PER_W)
        ]
        for cp in copies:
            cp.wait()
        pltpu.sync_copy(rows_v, out_hbm.at[pl.ds(base, _B_PER_W)])

    return _sc_gather


def kernel(x, embedding):
    b, h, w, c = x.shape
    x_flat = jnp.reshape(x, (b * h * w, c))
    idx = _encode(x_flat, embedding)
    idx2 = jnp.reshape(idx, (_IDX_ROWS, _IDX_COLS))
    quantized = _make_sc_gather()(embedding, idx2)
    quantized = quantized.astype(jnp.bfloat16).astype(jnp.float32)
    return jnp.reshape(quantized, (b, h, w, c))


# speedup vs baseline: 2.0588x; 1.2401x over previous
import functools

import jax
import jax.numpy as jnp
from jax import lax
from jax.experimental import pallas as pl
from jax.experimental.pallas import tpu as pltpu
from jax.experimental.pallas import tpu_sc as plsc

_N_TOK = 16384
_N_EMB = 8192
_D = 32
_TBLK = 4096
_NB = _N_TOK // _TBLK


_CH = 1024
_NCH = _N_EMB // _CH


def _argmin_body(x_ref, emb_ref, idx_ref, esq_ref, iota_ref, ebf_ref):
    @pl.when(pl.program_id(0) == 0)
    def _():
        emb = emb_ref[...]
        e2 = emb * emb
        ones = jnp.ones((1, _D), jnp.float32)
        esq_ref[...] = lax.dot_general(
            ones, e2, (((1,), (1,)), ((), ())),
            precision=lax.Precision.HIGHEST,
            preferred_element_type=jnp.float32,
        )
        iota_ref[...] = lax.broadcasted_iota(
            jnp.int32, (1, 128), 1).astype(jnp.float32)
        ebf_ref[...] = emb.astype(jnp.bfloat16)

    xb = x_ref[...]
    xm2 = (xb * -2.0).astype(jnp.bfloat16)
    xsq = jnp.sum(xb * xb, axis=1, keepdims=True)
    iota = iota_ref[...]
    bv = bi = None
    for c in range(_NCH):
        ec = ebf_ref[pl.ds(c * _CH, _CH), :]
        s2c = lax.dot_general(
            xm2, ec, (((1,), (1,)), ((), ())),
            preferred_element_type=jnp.float32,
        )
        for t in range(_CH // 128):
            j0 = c * _CH + t * 128
            d = ((xsq + esq_ref[:, pl.ds(j0, 128)])
                 + s2c[:, t * 128:(t + 1) * 128])
            idxrow = iota + jnp.float32(j0)
            if bv is None:
                bv = d
                bi = idxrow + jnp.zeros_like(d)
            else:
                lt = d < bv
                bv = jnp.minimum(bv, d)
                bi = jnp.where(lt, idxrow, bi)
    gm = jnp.min(bv, axis=1, keepdims=True)
    gi = jnp.min(jnp.where(bv == gm, bi, jnp.float32(_N_EMB)), axis=1)
    idx_ref[0, 0, :] = gi.astype(jnp.int32)


def _encode(x_flat, embedding):
    return pl.pallas_call(
        _argmin_body,
        grid=(_NB,),
        in_specs=[
            pl.BlockSpec((_TBLK, _D), lambda i: (i, 0)),
            pl.BlockSpec((_N_EMB, _D), lambda i: (0, 0)),
        ],
        out_specs=pl.BlockSpec((1, 1, _TBLK), lambda i: (i, 0, 0)),
        out_shape=jax.ShapeDtypeStruct((_NB, 1, _TBLK), jnp.int32),
        scratch_shapes=[pltpu.VMEM((1, _N_EMB), jnp.float32),
                        pltpu.VMEM((1, 128), jnp.float32),
                        pltpu.VMEM((_N_EMB, _D), jnp.bfloat16)],
    )(x_flat, embedding)


_IDX_COLS = 128
_IDX_ROWS = _N_TOK // _IDX_COLS
_NW = 32
_ROWS_PER_W = _IDX_ROWS // _NW
_B_PER_W = _N_TOK // _NW


@functools.cache
def _make_sc_gather():
    @functools.partial(
        pl.kernel,
        out_type=jax.ShapeDtypeStruct((_N_TOK, _D), jnp.float32),
        mesh=plsc.VectorSubcoreMesh(core_axis_name="c", subcore_axis_name="s"),
        scratch_types=[
            pltpu.VMEM((_ROWS_PER_W, _IDX_COLS), jnp.int32),
            pltpu.VMEM((_B_PER_W, _D), jnp.float32),
            pltpu.SemaphoreType.DMA,
        ],
        compiler_params=pltpu.CompilerParams(use_tc_tiling_on_sc=False),
    )
    def _sc_gather(table_hbm, idx_hbm, out_hbm, idx_v, rows_v, sem):
        wid = lax.axis_index("s") * 2 + lax.axis_index("c")
        base = wid * _B_PER_W
        pltpu.sync_copy(idx_hbm.at[pl.ds(wid * _ROWS_PER_W, _ROWS_PER_W)], idx_v)
        copies = [
            pltpu.async_copy(
                table_hbm.at[idx_v.at[j]],
                rows_v.at[pl.ds(j * _IDX_COLS, _IDX_COLS)],
                sem,
            )
            for j in range(_ROWS_PER_W)
        ]
        for cp in copies:
            cp.wait()
        pltpu.sync_copy(rows_v, out_hbm.at[pl.ds(base, _B_PER_W)])

    return _sc_gather


def kernel(x, embedding):
    b, h, w, c = x.shape
    x_flat = jnp.reshape(x, (b * h * w, c))
    idx = _encode(x_flat, embedding)
    idx2 = jnp.reshape(idx, (_IDX_ROWS, _IDX_COLS))
    quantized = _make_sc_gather()(embedding, idx2)
    quantized = quantized.astype(jnp.bfloat16).astype(jnp.float32)
    return jnp.reshape(quantized, (b, h, w, c))
